# baseline (speedup 1.0000x reference)
import functools

import jax
import jax.numpy as jnp
from jax import lax
from jax.experimental import pallas as pl
from jax.experimental.pallas import tpu as pltpu
from jax.experimental.pallas import tpu_sc as plsc

N_FACTORS = 64
BATCH = 16384
NUM_WORKERS = 32
B_PER_W = BATCH // NUM_WORKERS
IDX_CHUNK = 128
N_CHUNKS = B_PER_W // IDX_CHUNK
N_GROUPS = B_PER_W // 16
W_ROWS = 8
G_ROWS = 16 * W_ROWS


def _mf_body(uids_hbm, iids_hbm, uf_hbm, if_hbm, ub_hbm, ib_hbm, gb_hbm,
             out_hbm,
             idx_u, idx_i, urows0, urows1, irows0, irows1,
             bu0, bu1, bi0, bi1, gb_v, out_v, sem0, sem1):
    wid = lax.axis_index("s") * 2 + lax.axis_index("c")
    base = wid * B_PER_W

    for j in range(N_CHUNKS):
        src = pl.ds(base + j * IDX_CHUNK, IDX_CHUNK)
        dst = pl.ds(j * IDX_CHUNK, IDX_CHUNK)
        pltpu.sync_copy(uids_hbm.at[src], idx_u.at[dst])
        pltpu.sync_copy(iids_hbm.at[src], idx_i.at[dst])
    pltpu.sync_copy(gb_hbm, gb_v)
    gb = gb_v[...]

    def issue(g, urows, irows, bu, bi, sem):
        col0 = g * 16
        vu = idx_u[pl.ds(col0, 16)]
        vi = idx_i[pl.ds(col0, 16)]
        for l in range(16):
            ru = (vu[l] >> 3) << 3
            ri = (vi[l] >> 3) << 3
            ru = pl.multiple_of(ru, 8)
            ri = pl.multiple_of(ri, 8)
            dstw = pl.ds(l * W_ROWS, W_ROWS)
            pltpu.async_copy(uf_hbm.at[pl.ds(ru, W_ROWS), :],
                             urows.at[dstw, :], sem)
            pltpu.async_copy(if_hbm.at[pl.ds(ri, W_ROWS), :],
                             irows.at[dstw, :], sem)
            pltpu.async_copy(ub_hbm.at[pl.ds(ru, W_ROWS)], bu.at[dstw], sem)
            pltpu.async_copy(ib_hbm.at[pl.ds(ri, W_ROWS)], bi.at[dstw], sem)

    def drain(urows, irows, bu, bi, sem):
        pltpu.make_async_copy(uf_hbm.at[pl.ds(0, G_ROWS), :], urows, sem).wait()
        pltpu.make_async_copy(if_hbm.at[pl.ds(0, G_ROWS), :], irows, sem).wait()
        pltpu.make_async_copy(ub_hbm.at[pl.ds(0, G_ROWS)], bu, sem).wait()
        pltpu.make_async_copy(ib_hbm.at[pl.ds(0, G_ROWS)], bi, sem).wait()

    lane8 = lax.iota(jnp.int32, 16) * W_ROWS

    def compute(g, urows, irows, bu, bi):
        col0 = g * 16
        vu = idx_u[pl.ds(col0, 16)]
        vi = idx_i[pl.ds(col0, 16)]
        rows_u = lane8 + (vu & 7)
        rows_i = lane8 + (vi & 7)
        acc = (plsc.load_gather(bu, [rows_u])
               + plsc.load_gather(bi, [rows_i]) + gb)
        for d in range(N_FACTORS):
            drow = jnp.full((16,), d, jnp.int32)
            u = plsc.load_gather(urows, [rows_u, drow])
            v = plsc.load_gather(irows, [rows_i, drow])
            acc = acc + u * v
        out_v[pl.ds(col0, 16)] = acc

    issue(0, urows0, irows0, bu0, bi0, sem0)

    def pair_body(t, carry):
        g0 = t * 2
        g1 = g0 + 1
        issue(g1, urows1, irows1, bu1, bi1, sem1)
        drain(urows0, irows0, bu0, bi0, sem0)
        compute(g0, urows0, irows0, bu0, bi0)

        @pl.when(t < (N_GROUPS // 2 - 1))
        def _():
            issue(g0 + 2, urows0, irows0, bu0, bi0, sem0)

        drain(urows1, irows1, bu1, bi1, sem1)
        compute(g1, urows1, irows1, bu1, bi1)
        return carry

    lax.fori_loop(0, N_GROUPS // 2, pair_body, 0)

    pltpu.sync_copy(out_v, out_hbm.at[pl.ds(base, B_PER_W)])


_mf_kernel = functools.partial(
    pl.kernel,
    mesh=plsc.VectorSubcoreMesh(core_axis_name="c", subcore_axis_name="s"),
    out_type=jax.ShapeDtypeStruct((BATCH,), jnp.float32),
    compiler_params=pltpu.CompilerParams(needs_layout_passes=False,
                                         use_tc_tiling_on_sc=True),
    scratch_types=[
        pltpu.VMEM((B_PER_W,), jnp.int32),
        pltpu.VMEM((B_PER_W,), jnp.int32),
        pltpu.VMEM((G_ROWS, N_FACTORS), jnp.float32),
        pltpu.VMEM((G_ROWS, N_FACTORS), jnp.float32),
        pltpu.VMEM((G_ROWS, N_FACTORS), jnp.float32),
        pltpu.VMEM((G_ROWS, N_FACTORS), jnp.float32),
        pltpu.VMEM((G_ROWS,), jnp.float32),
        pltpu.VMEM((G_ROWS,), jnp.float32),
        pltpu.VMEM((G_ROWS,), jnp.float32),
        pltpu.VMEM((G_ROWS,), jnp.float32),
        pltpu.VMEM((16,), jnp.float32),
        pltpu.VMEM((B_PER_W,), jnp.float32),
        pltpu.SemaphoreType.DMA,
        pltpu.SemaphoreType.DMA,
    ],
)(_mf_body)


@jax.jit
def kernel(user_ids, item_ids, user_factors, item_factors, user_bias,
           item_bias, global_bias):
    gb16 = jnp.broadcast_to(global_bias.reshape(()), (16,))
    return _mf_kernel(user_ids.astype(jnp.int32), item_ids.astype(jnp.int32),
                      user_factors, item_factors,
                      user_bias.reshape(-1), item_bias.reshape(-1), gb16)
